# Initial kernel scaffold; baseline (speedup 1.0000x reference)
import jax
import jax.numpy as jnp
from jax.experimental import pallas as pl


def kernel(x):
    raise NotImplementedError("write your pallas kernel here")


# speedup vs baseline: 1.6409x; 1.6409x over previous
import jax
import jax.numpy as jnp
from jax.experimental import pallas as pl
from jax.experimental.pallas import tpu as pltpu

_B, _N, _C = 4, 8192, 2048
_R = 1024
_NCHUNK = _N // _R


def _argmin_body(x_ref, o_ref, val_ref, idx_ref):
    c = pl.program_id(1)
    chunk = x_ref[0]
    lmin = jnp.min(chunk, axis=0)
    iota = jax.lax.broadcasted_iota(jnp.int32, (_R, _C), 0)
    masked = jnp.where(chunk == lmin[None, :], iota, _N)
    larg = jnp.min(masked, axis=0) + c * _R

    @pl.when(c == 0)
    def _():
        val_ref[0] = lmin
        idx_ref[0] = larg

    @pl.when(c > 0)
    def _():
        better = lmin < val_ref[0]
        val_ref[0] = jnp.where(better, lmin, val_ref[0])
        idx_ref[0] = jnp.where(better, larg, idx_ref[0])

    @pl.when(c == _NCHUNK - 1)
    def _():
        o_ref[0, 0] = idx_ref[0]


def kernel(x):
    out = pl.pallas_call(
        _argmin_body,
        grid=(_B, _NCHUNK),
        in_specs=[pl.BlockSpec((1, _R, _C), lambda b, c: (b, c, 0))],
        out_specs=pl.BlockSpec((1, 1, _C), lambda b, c: (b, 0, 0)),
        out_shape=jax.ShapeDtypeStruct((_B, 1, _C), jnp.int32),
        scratch_shapes=[
            pltpu.VMEM((1, _C), jnp.float32),
            pltpu.VMEM((1, _C), jnp.int32),
        ],
        compiler_params=pltpu.CompilerParams(
            dimension_semantics=("arbitrary", "arbitrary")
        ),
    )(x)
    return out.reshape(_B, _C).astype(jnp.int64)
